# Initial kernel scaffold; baseline (speedup 1.0000x reference)
import functools

import jax
import jax.numpy as jnp
from jax import lax
from jax.experimental import pallas as pl
from jax.experimental.pallas import tpu as pltpu
from jax.experimental.pallas import tpu_sc as plsc

_F32 = jnp.float32
_HIGH = jax.lax.Precision.HIGHEST


def _dot(a, b):
    return jnp.dot(a, b, preferred_element_type=_F32, precision=_HIGH)


def _ab_body(x_ref, wa_ref, wb_ref, a_ref, b_ref):
    xb = x_ref[...]
    a_ref[...] = _dot(xb, wa_ref[...])
    b_ref[...] = _dot(xb, wb_ref[...])


def _c_body(ea_ref, we_ref, b1_ref, c_ref):
    c_ref[...] = _dot(ea_ref[...], we_ref[...]) + b1_ref[...]


def _node_body(a1_ref, a2_ref, x_ref, pp_ref, w1x_ref, w1e_ref, b1_ref,
               w2_ref, b2_ref, nx_ref, np_ref):
    a1 = a1_ref[0] + a1_ref[1]
    a2 = a2_ref[0] + a2_ref[1]
    denom = jnp.maximum(a1[:, 3:4], 1.0)
    np_ref[...] = pp_ref[...] + a1 / denom
    agg = a2 / denom
    hmid = jnp.maximum(
        _dot(x_ref[...], w1x_ref[...]) + _dot(agg, w1e_ref[...]) + b1_ref[...],
        0.0)
    nx_ref[...] = _dot(hmid, w2_ref[...]) + b2_ref[...]


def _build_sc_edge(N, E, DF, DE, H, K, NC, NS):
    EPW = E // (NC * NS)
    NCHUNK = EPW // K
    ROWS = N // NS
    L = 16

    def c16(v, dtype=jnp.int32):
        return jnp.full((L,), v, dtype)

    def body(a_h, b_h, p_h, c_h, ea_h, src_h, dst_h, w2_h, w1d_h, b2_h,
             acc1_o, acc2_o,
             zbuf, src_i, dst_i, a_v, b_v, c_v, ps_v, pd_v, ea_v, pay_v,
             w2_v, w1d_v, b2_v, acc1_sh, acc2_sh,
             sem_a, sem_b, sem_ps, sem_pd):
        cid = lax.axis_index("c")
        sid = lax.axis_index("s")
        wid = cid * NS + sid

        zf = jnp.zeros((L,), _F32)

        def _zero_row(i, carry):
            zbuf[i] = zf
            return carry
        lax.fori_loop(0, ROWS, _zero_row, 0)

        def _zero_pay(i, carry):
            pay_v[i] = zf
            return carry
        lax.fori_loop(0, K, _zero_pay, 0)

        r0 = sid * ROWS
        pltpu.sync_copy(zbuf, acc1_sh.at[pl.ds(r0, ROWS)])
        pltpu.sync_copy(zbuf, acc2_sh.at[pl.ds(r0, ROWS)])
        pltpu.sync_copy(w2_h, w2_v)
        pltpu.sync_copy(w1d_h, w1d_v)
        pltpu.sync_copy(b2_h, b2_v)
        plsc.subcore_barrier()

        lanes = jnp.arange(L, dtype=jnp.int32)
        ebase = wid * EPW

        def chunk(ci, carry):
            eb = ebase + ci * K
            pltpu.sync_copy(src_h.at[pl.ds(eb, K)], src_i)
            pltpu.sync_copy(dst_h.at[pl.ds(eb, K)], dst_i)
            cp_a = pltpu.async_copy(a_h.at[src_i], a_v, sem_a)
            cp_b = pltpu.async_copy(b_h.at[dst_i], b_v, sem_b)
            cp_ps = pltpu.async_copy(p_h.at[src_i], ps_v, sem_ps)
            cp_pd = pltpu.async_copy(p_h.at[dst_i], pd_v, sem_pd)
            pltpu.sync_copy(c_h.at[pl.ds(eb, K)], c_v)
            pltpu.sync_copy(ea_h.at[pl.ds(eb, K)], ea_v)
            cp_a.wait()
            cp_b.wait()
            cp_ps.wait()
            cp_pd.wait()

            for g in range(K // L):
                rows = lanes + (g * L)
                pxs = plsc.load_gather(ps_v, [rows, c16(0)])
                pys = plsc.load_gather(ps_v, [rows, c16(1)])
                pzs = plsc.load_gather(ps_v, [rows, c16(2)])
                pxd = plsc.load_gather(pd_v, [rows, c16(0)])
                pyd = plsc.load_gather(pd_v, [rows, c16(1)])
                pzd = plsc.load_gather(pd_v, [rows, c16(2)])
                rx = pxd - pxs
                ry = pyd - pys
                rz = pzd - pzs
                d2 = rx * rx + ry * ry + rz * rz
                ib = plsc.bitcast(d2, jnp.int32)
                r = plsc.bitcast(c16(0x5F3759DF) - (ib >> 1), _F32)
                half = d2 * 0.5
                for _ in range(3):
                    r = r * (1.5 - half * r * r)
                dist = d2 * r

                def kstep(k, acc):
                    colk = jnp.full((L,), k, jnp.int32)
                    va = plsc.load_gather(a_v, [rows, colk])
                    vb = plsc.load_gather(b_v, [rows, colk])
                    vc = plsc.load_gather(c_v, [rows, colk])
                    pre = va + vb + vc + dist * w1d_v[k]
                    hp = jnp.maximum(pre, 0.0)
                    return acc + hp * w2_v[k]

                acc = lax.fori_loop(0, H, kstep, jnp.zeros((L,), _F32))
                w = acc + b2_v[0]
                plsc.store_scatter(pay_v, [rows, c16(0)], w * rx)
                plsc.store_scatter(pay_v, [rows, c16(1)], w * ry)
                plsc.store_scatter(pay_v, [rows, c16(2)], w * rz)
                plsc.store_scatter(pay_v, [rows, c16(3)], c16(1.0, _F32))

            pltpu.sync_copy(pay_v, acc1_sh.at[dst_i], add=True)
            pltpu.sync_copy(ea_v, acc2_sh.at[dst_i], add=True)
            return carry

        lax.fori_loop(0, NCHUNK, chunk, 0)
        plsc.subcore_barrier()

        pltpu.sync_copy(acc1_sh.at[pl.ds(r0, ROWS)], zbuf)
        pltpu.sync_copy(zbuf, acc1_o.at[cid, pl.ds(r0, ROWS)])
        pltpu.sync_copy(acc2_sh.at[pl.ds(r0, ROWS)], zbuf)
        pltpu.sync_copy(zbuf, acc2_o.at[cid, pl.ds(r0, ROWS)])

    kern = pl.kernel(
        body,
        out_type=[jax.ShapeDtypeStruct((NC, N, 16), _F32),
                  jax.ShapeDtypeStruct((NC, N, DE), _F32)],
        mesh=plsc.VectorSubcoreMesh(core_axis_name="c", subcore_axis_name="s"),
        scratch_types=[
            pltpu.VMEM((ROWS, 16), _F32),
            pltpu.VMEM((K,), jnp.int32),
            pltpu.VMEM((K,), jnp.int32),
            pltpu.VMEM((K, H), _F32),
            pltpu.VMEM((K, H), _F32),
            pltpu.VMEM((K, H), _F32),
            pltpu.VMEM((K, 16), _F32),
            pltpu.VMEM((K, 16), _F32),
            pltpu.VMEM((K, DE), _F32),
            pltpu.VMEM((K, 16), _F32),
            pltpu.VMEM((H,), _F32),
            pltpu.VMEM((H,), _F32),
            pltpu.VMEM((1,), _F32),
            pltpu.VMEM_SHARED((N, 16), _F32),
            pltpu.VMEM_SHARED((N, DE), _F32),
            pltpu.SemaphoreType.DMA,
            pltpu.SemaphoreType.DMA,
            pltpu.SemaphoreType.DMA,
            pltpu.SemaphoreType.DMA,
        ],
    )
    return kern


def kernel(x, edge_index, edge_attr, pos, W1c, b1c, W2c, b2c,
           W1n, b1n, W2n, b2n):
    N, DF = x.shape
    E = edge_index.shape[1]
    DE = edge_attr.shape[1]
    H = W1c.shape[1]
    NC, NS = 2, 16
    K = 80
    assert E % (NC * NS * K) == 0 and N % NS == 0

    W1c_src = W1c[:DF]
    W1c_dst = W1c[DF:2 * DF]
    W1c_e = W1c[2 * DF:2 * DF + DE]
    w1d = W1c[2 * DF + DE]
    b1c2 = b1c.reshape(1, H)
    w2c = W2c.reshape(H)
    pos_pad = jnp.pad(pos, ((0, 0), (0, 13)))
    src = edge_index[0]
    dst = edge_index[1]

    NB = 1000
    A, B = pl.pallas_call(
        _ab_body,
        grid=(N // NB,),
        in_specs=[pl.BlockSpec((NB, DF), lambda i: (i, 0)),
                  pl.BlockSpec((DF, H), lambda i: (0, 0)),
                  pl.BlockSpec((DF, H), lambda i: (0, 0))],
        out_specs=[pl.BlockSpec((NB, H), lambda i: (i, 0)),
                   pl.BlockSpec((NB, H), lambda i: (i, 0))],
        out_shape=[jax.ShapeDtypeStruct((N, H), _F32),
                   jax.ShapeDtypeStruct((N, H), _F32)],
    )(x, W1c_src, W1c_dst)

    EB = 4000
    C = pl.pallas_call(
        _c_body,
        grid=(E // EB,),
        in_specs=[pl.BlockSpec((EB, DE), lambda i: (i, 0)),
                  pl.BlockSpec((DE, H), lambda i: (0, 0)),
                  pl.BlockSpec((1, H), lambda i: (0, 0))],
        out_specs=pl.BlockSpec((EB, H), lambda i: (i, 0)),
        out_shape=jax.ShapeDtypeStruct((E, H), _F32),
    )(edge_attr, W1c_e, b1c2)

    sc = _build_sc_edge(N, E, DF, DE, H, K, NC, NS)
    acc1, acc2 = sc(A, B, pos_pad, C, edge_attr, src, dst, w2c, w1d, b2c)

    W1n_x = W1n[:DF]
    W1n_e = W1n[DF:]
    b1n2 = b1n.reshape(1, H)
    b2n2 = b2n.reshape(1, DF)
    new_x, new_pos_pad = pl.pallas_call(
        _node_body,
        grid=(N // NB,),
        in_specs=[pl.BlockSpec((NC, NB, 16), lambda i: (0, i, 0)),
                  pl.BlockSpec((NC, NB, DE), lambda i: (0, i, 0)),
                  pl.BlockSpec((NB, DF), lambda i: (i, 0)),
                  pl.BlockSpec((NB, 16), lambda i: (i, 0)),
                  pl.BlockSpec((DF, H), lambda i: (0, 0)),
                  pl.BlockSpec((DE, H), lambda i: (0, 0)),
                  pl.BlockSpec((1, H), lambda i: (0, 0)),
                  pl.BlockSpec((H, DF), lambda i: (0, 0)),
                  pl.BlockSpec((1, DF), lambda i: (0, 0))],
        out_specs=[pl.BlockSpec((NB, DF), lambda i: (i, 0)),
                   pl.BlockSpec((NB, 16), lambda i: (i, 0))],
        out_shape=[jax.ShapeDtypeStruct((N, DF), _F32),
                   jax.ShapeDtypeStruct((N, 16), _F32)],
    )(acc1, acc2, x, pos_pad, W1n_x, W1n_e, b1n2, W2n, b2n2)

    return (new_x, new_pos_pad[:, :3])


# speedup vs baseline: 1.7035x; 1.7035x over previous
import functools

import jax
import jax.numpy as jnp
from jax import lax
from jax.experimental import pallas as pl
from jax.experimental.pallas import tpu as pltpu
from jax.experimental.pallas import tpu_sc as plsc

_F32 = jnp.float32
_HIGH = jax.lax.Precision.HIGHEST


def _dot(a, b):
    return jnp.dot(a, b, preferred_element_type=_F32, precision=_HIGH)


def _ab_body(x_ref, wa_ref, wb_ref, a_ref, b_ref):
    xb = x_ref[...]
    a_ref[...] = _dot(xb, wa_ref[...])
    b_ref[...] = _dot(xb, wb_ref[...])


def _c_body(ea_ref, we_ref, b1_ref, c_ref):
    c_ref[...] = _dot(ea_ref[...], we_ref[...]) + b1_ref[...]


def _node_body(acc_ref, x_ref, pp_ref, w1x_ref, w1e_ref, b1_ref,
               w2_ref, b2_ref, nx_ref, np_ref):
    a = acc_ref[0] + acc_ref[1]
    denom = jnp.maximum(a[:, 3:4], 1.0)
    np_ref[...] = pp_ref[...] + a[:, 0:16] / denom
    agg = a[:, 4:20] / denom
    hmid = jnp.maximum(
        _dot(x_ref[...], w1x_ref[...]) + _dot(agg, w1e_ref[...]) + b1_ref[...],
        0.0)
    nx_ref[...] = _dot(hmid, w2_ref[...]) + b2_ref[...]


def _build_sc_edge(N, NP, E, DF, DE, H, K, NC, NS):
    EPW = E // (NC * NS)
    NCHUNK = EPW // K
    ROWS = NP // NS
    L = 16

    def c16(v, dtype=jnp.int32):
        return jnp.full((L,), v, dtype)

    def body(a_h, b_h, px_h, py_h, pz_h, c_h, ea_h, src_h, dst_h,
             w2_h, w1d_h, b2_h, acc_o,
             src_i, dst_i, a_v, b_v, c_v, pay_v, ea_v,
             psx, psy, psz, pdx, pdy, pdz,
             wtmp, w2_v, w1d_v, b2_v, acc_sh,
             sem_a, sem_b, sem_p):
        cid = lax.axis_index("c")
        sid = lax.axis_index("s")
        wid = cid * NS + sid

        zf = jnp.zeros((L,), _F32)

        def _zero_pay(i, carry):
            for c in range(H // L):
                pay_v[i, pl.ds(c * L, L)] = zf
            return carry
        lax.fori_loop(0, K, _zero_pay, 0)

        r0 = sid * ROWS

        def _zero_acc(j, carry):
            pltpu.sync_copy(pay_v, acc_sh.at[pl.ds(r0 + j * K, K)])
            return carry
        lax.fori_loop(0, ROWS // K, _zero_acc, 0)

        pltpu.sync_copy(w2_h, wtmp)
        for gg in range(H // L):
            v = wtmp[pl.ds(gg * L, L)]
            for j in range(L):
                w2_v[gg * L + j] = v[j]
        pltpu.sync_copy(w1d_h, wtmp)
        for gg in range(H // L):
            v = wtmp[pl.ds(gg * L, L)]
            for j in range(L):
                w1d_v[gg * L + j] = v[j]
        pltpu.sync_copy(b2_h, wtmp.at[pl.ds(0, 1)])
        vb2 = wtmp[pl.ds(0, L)]
        b2_v[0] = vb2[0]
        plsc.subcore_barrier()

        lanes = jnp.arange(L, dtype=jnp.int32)
        ebase = wid * EPW

        def chunk(ci, carry):
            eb = ebase + ci * K
            pltpu.sync_copy(src_h.at[pl.ds(eb, K)], src_i)
            pltpu.sync_copy(dst_h.at[pl.ds(eb, K)], dst_i)
            cp_a = pltpu.async_copy(a_h.at[src_i], a_v, sem_a)
            cp_b = pltpu.async_copy(b_h.at[dst_i], b_v, sem_b)
            cp1 = pltpu.async_copy(px_h.at[src_i], psx, sem_p)
            cp2 = pltpu.async_copy(py_h.at[src_i], psy, sem_p)
            cp3 = pltpu.async_copy(pz_h.at[src_i], psz, sem_p)
            cp4 = pltpu.async_copy(px_h.at[dst_i], pdx, sem_p)
            cp5 = pltpu.async_copy(py_h.at[dst_i], pdy, sem_p)
            cp6 = pltpu.async_copy(pz_h.at[dst_i], pdz, sem_p)
            pltpu.sync_copy(c_h.at[pl.ds(eb, K)], c_v)
            pltpu.sync_copy(ea_h.at[pl.ds(eb * DE, K * DE)], ea_v)
            cp1.wait()
            cp2.wait()
            cp3.wait()
            cp4.wait()
            cp5.wait()
            cp6.wait()
            cp_a.wait()
            cp_b.wait()

            for g in range(K // L):
                rows = lanes + (g * L)
                gl = g * L
                rx = pdx[pl.ds(gl, L)] - psx[pl.ds(gl, L)]
                ry = pdy[pl.ds(gl, L)] - psy[pl.ds(gl, L)]
                rz = pdz[pl.ds(gl, L)] - psz[pl.ds(gl, L)]
                d2 = rx * rx + ry * ry + rz * rz
                ib = plsc.bitcast(d2, jnp.int32)
                r = plsc.bitcast(c16(0x5F3759DF) - (ib >> 1), _F32)
                half = d2 * 0.5
                for _ in range(3):
                    r = r * (1.5 - half * r * r)
                dist = d2 * r

                def kstep(k, acc):
                    colk = jnp.full((L,), k, jnp.int32)
                    va = plsc.load_gather(a_v, [rows, colk])
                    vb = plsc.load_gather(b_v, [rows, colk])
                    vc = plsc.load_gather(c_v, [rows, colk])
                    pre = va + vb + vc + dist * w1d_v[k]
                    hp = jnp.maximum(pre, 0.0)
                    return acc + hp * w2_v[k]

                acc = lax.fori_loop(0, H, kstep, jnp.zeros((L,), _F32))
                w = acc + b2_v[0]
                plsc.store_scatter(pay_v, [rows, c16(0)], w * rx)
                plsc.store_scatter(pay_v, [rows, c16(1)], w * ry)
                plsc.store_scatter(pay_v, [rows, c16(2)], w * rz)
                plsc.store_scatter(pay_v, [rows, c16(3)], c16(1.0, _F32))
                eaidx = rows * DE
                for j in range(DE):
                    v = plsc.load_gather(ea_v, [eaidx + j])
                    plsc.store_scatter(pay_v, [rows, c16(4 + j)], v)

            pltpu.sync_copy(pay_v, acc_sh.at[dst_i], add=True)
            return carry

        lax.fori_loop(0, NCHUNK, chunk, 0)
        plsc.subcore_barrier()

        def _drain(j, carry):
            pltpu.sync_copy(acc_sh.at[pl.ds(r0 + j * K, K)], pay_v)
            pltpu.sync_copy(pay_v, acc_o.at[cid, pl.ds(r0 + j * K, K)])
            return carry
        lax.fori_loop(0, ROWS // K, _drain, 0)

    kern = pl.kernel(
        body,
        out_type=[jax.ShapeDtypeStruct((NC, NP, H), _F32)],
        mesh=plsc.VectorSubcoreMesh(core_axis_name="c", subcore_axis_name="s"),
        scratch_types=[
            pltpu.VMEM((K,), jnp.int32),
            pltpu.VMEM((K,), jnp.int32),
            pltpu.VMEM((K, H), _F32),
            pltpu.VMEM((K, H), _F32),
            pltpu.VMEM((K, H), _F32),
            pltpu.VMEM((K, H), _F32),
            pltpu.VMEM((K * DE,), _F32),
            pltpu.VMEM((K,), _F32),
            pltpu.VMEM((K,), _F32),
            pltpu.VMEM((K,), _F32),
            pltpu.VMEM((K,), _F32),
            pltpu.VMEM((K,), _F32),
            pltpu.VMEM((K,), _F32),
            pltpu.VMEM((H,), _F32),
            pltpu.SMEM((H,), _F32),
            pltpu.SMEM((H,), _F32),
            pltpu.SMEM((1,), _F32),
            pltpu.VMEM_SHARED((NP, H), _F32),
            pltpu.SemaphoreType.DMA,
            pltpu.SemaphoreType.DMA,
            pltpu.SemaphoreType.DMA,
        ],
        compiler_params=pltpu.CompilerParams(needs_layout_passes=False),
    )
    return kern


def kernel(x, edge_index, edge_attr, pos, W1c, b1c, W2c, b2c,
           W1n, b1n, W2n, b2n):
    N, DF = x.shape
    E = edge_index.shape[1]
    DE = edge_attr.shape[1]
    H = W1c.shape[1]
    NC, NS = 2, 16
    K = 80
    assert E % (NC * NS * K) == 0 and N % NS == 0

    W1c_src = W1c[:DF]
    W1c_dst = W1c[DF:2 * DF]
    W1c_e = W1c[2 * DF:2 * DF + DE]
    w1d = W1c[2 * DF + DE]
    b1c2 = b1c.reshape(1, H)
    w2c = W2c.reshape(H)
    pos_pad = jnp.pad(pos, ((0, 0), (0, 13)))
    posx = pos[:, 0]
    posy = pos[:, 1]
    posz = pos[:, 2]
    ea_flat = edge_attr.reshape(-1)
    src = edge_index[0]
    dst = edge_index[1]

    NB = 1000
    A, B = pl.pallas_call(
        _ab_body,
        grid=(N // NB,),
        in_specs=[pl.BlockSpec((NB, DF), lambda i: (i, 0)),
                  pl.BlockSpec((DF, H), lambda i: (0, 0)),
                  pl.BlockSpec((DF, H), lambda i: (0, 0))],
        out_specs=[pl.BlockSpec((NB, H), lambda i: (i, 0)),
                   pl.BlockSpec((NB, H), lambda i: (i, 0))],
        out_shape=[jax.ShapeDtypeStruct((N, H), _F32),
                   jax.ShapeDtypeStruct((N, H), _F32)],
    )(x, W1c_src, W1c_dst)

    EB = 4000
    C = pl.pallas_call(
        _c_body,
        grid=(E // EB,),
        in_specs=[pl.BlockSpec((EB, DE), lambda i: (i, 0)),
                  pl.BlockSpec((DE, H), lambda i: (0, 0)),
                  pl.BlockSpec((1, H), lambda i: (0, 0))],
        out_specs=pl.BlockSpec((EB, H), lambda i: (i, 0)),
        out_shape=jax.ShapeDtypeStruct((E, H), _F32),
    )(edge_attr, W1c_e, b1c2)

    NP = ((N + NS * K - 1) // (NS * K)) * NS * K
    sc = _build_sc_edge(N, NP, E, DF, DE, H, K, NC, NS)
    (acc,) = sc(A, B, posx, posy, posz, C, ea_flat, src, dst, w2c, w1d, b2c)

    W1n_x = W1n[:DF]
    W1n_e = W1n[DF:]
    b1n2 = b1n.reshape(1, H)
    b2n2 = b2n.reshape(1, DF)
    new_x, new_pos_pad = pl.pallas_call(
        _node_body,
        grid=(N // NB,),
        in_specs=[pl.BlockSpec((NC, NB, H), lambda i: (0, i, 0)),
                  pl.BlockSpec((NB, DF), lambda i: (i, 0)),
                  pl.BlockSpec((NB, 16), lambda i: (i, 0)),
                  pl.BlockSpec((DF, H), lambda i: (0, 0)),
                  pl.BlockSpec((DE, H), lambda i: (0, 0)),
                  pl.BlockSpec((1, H), lambda i: (0, 0)),
                  pl.BlockSpec((H, DF), lambda i: (0, 0)),
                  pl.BlockSpec((1, DF), lambda i: (0, 0))],
        out_specs=[pl.BlockSpec((NB, DF), lambda i: (i, 0)),
                   pl.BlockSpec((NB, 16), lambda i: (i, 0))],
        out_shape=[jax.ShapeDtypeStruct((N, DF), _F32),
                   jax.ShapeDtypeStruct((N, 16), _F32)],
    )(acc, x, pos_pad, W1n_x, W1n_e, b1n2, W2n, b2n2)

    return (new_x, new_pos_pad[:, :3])


# speedup vs baseline: 1.9098x; 1.1211x over previous
import functools

import jax
import jax.numpy as jnp
from jax import lax
from jax.experimental import pallas as pl
from jax.experimental.pallas import tpu as pltpu
from jax.experimental.pallas import tpu_sc as plsc

_F32 = jnp.float32
_HIGH = jax.lax.Precision.HIGHEST


def _dot(a, b):
    return jnp.dot(a, b, preferred_element_type=_F32, precision=_HIGH)


def _ab_body(x_ref, wa_ref, wb_ref, a_ref, b_ref):
    xb = x_ref[...]
    a_ref[...] = _dot(xb, wa_ref[...])
    b_ref[...] = _dot(xb, wb_ref[...])


def _c_body(ea_ref, we_ref, b1_ref, c_ref):
    c_ref[...] = _dot(ea_ref[...], we_ref[...]) + b1_ref[...]


def _node_body(acc_ref, x_ref, pp_ref, w1x_ref, w1e_ref, b1_ref,
               w2_ref, b2_ref, nx_ref, np_ref):
    a = acc_ref[0] + acc_ref[1]
    denom = jnp.maximum(a[:, 3:4], 1.0)
    np_ref[...] = pp_ref[...] + a[:, 0:16] / denom
    agg = a[:, 4:20] / denom
    hmid = jnp.maximum(
        _dot(x_ref[...], w1x_ref[...]) + _dot(agg, w1e_ref[...]) + b1_ref[...],
        0.0)
    nx_ref[...] = _dot(hmid, w2_ref[...]) + b2_ref[...]


def _build_sc_edge(N, NP, E, DF, DE, H, K, NC, NS):
    EPW = E // (NC * NS)
    NCHUNK = EPW // K
    ROWS = NP // NS
    L = 16

    def c16(v, dtype=jnp.int32):
        return jnp.full((L,), v, dtype)

    def body(a_h, b_h, px_h, py_h, pz_h, c_h, ea_h, src_h, dst_h,
             w2_h, w1d_h, b2_h, acc_o,
             src_i, dst_i, a_v, b_v, c_v, pay_v, ea_v,
             psx, psy, psz, pdx, pdy, pdz,
             wtmp, w2_v, w1d_v, b2_v, acc_sh,
             sem_a, sem_b, sem_p):
        cid = lax.axis_index("c")
        sid = lax.axis_index("s")
        wid = cid * NS + sid

        zf = jnp.zeros((L,), _F32)

        def _zero_pay(i, carry):
            for c in range(H // L):
                pay_v[i, pl.ds(c * L, L)] = zf
            return carry
        lax.fori_loop(0, K, _zero_pay, 0)

        r0 = sid * ROWS

        def _zero_acc(j, carry):
            pltpu.sync_copy(pay_v, acc_sh.at[pl.ds(r0 + j * K, K)])
            return carry
        lax.fori_loop(0, ROWS // K, _zero_acc, 0)

        pltpu.sync_copy(w2_h, wtmp)
        for gg in range(H // L):
            v = wtmp[pl.ds(gg * L, L)]
            for j in range(L):
                w2_v[gg * L + j] = v[j]
        pltpu.sync_copy(w1d_h, wtmp)
        for gg in range(H // L):
            v = wtmp[pl.ds(gg * L, L)]
            for j in range(L):
                w1d_v[gg * L + j] = v[j]
        pltpu.sync_copy(b2_h, wtmp.at[pl.ds(0, 1)])
        vb2 = wtmp[pl.ds(0, L)]
        b2_v[0] = vb2[0]
        plsc.subcore_barrier()

        lanes = jnp.arange(L, dtype=jnp.int32)
        ebase = wid * EPW

        def chunk(ci, carry):
            eb = ebase + ci * K
            pltpu.sync_copy(src_h.at[pl.ds(eb, K)], src_i)
            pltpu.sync_copy(dst_h.at[pl.ds(eb, K)], dst_i)
            cp_a = pltpu.async_copy(a_h.at[src_i], a_v, sem_a)
            cp_b = pltpu.async_copy(b_h.at[dst_i], b_v, sem_b)
            cp1 = pltpu.async_copy(px_h.at[src_i], psx, sem_p)
            cp2 = pltpu.async_copy(py_h.at[src_i], psy, sem_p)
            cp3 = pltpu.async_copy(pz_h.at[src_i], psz, sem_p)
            cp4 = pltpu.async_copy(px_h.at[dst_i], pdx, sem_p)
            cp5 = pltpu.async_copy(py_h.at[dst_i], pdy, sem_p)
            cp6 = pltpu.async_copy(pz_h.at[dst_i], pdz, sem_p)
            pltpu.sync_copy(c_h.at[pl.ds(eb, K)], c_v)
            pltpu.sync_copy(ea_h.at[pl.ds(eb * DE, K * DE)], ea_v)
            cp1.wait()
            cp2.wait()
            cp3.wait()
            cp4.wait()
            cp5.wait()
            cp6.wait()
            cp_a.wait()
            cp_b.wait()

            for g in range(K // L):
                rows = lanes + (g * L)
                gl = g * L
                rx = pdx[pl.ds(gl, L)] - psx[pl.ds(gl, L)]
                ry = pdy[pl.ds(gl, L)] - psy[pl.ds(gl, L)]
                rz = pdz[pl.ds(gl, L)] - psz[pl.ds(gl, L)]
                d2 = rx * rx + ry * ry + rz * rz
                ib = plsc.bitcast(d2, jnp.int32)
                r = plsc.bitcast(c16(0x5F3759DF) - (ib >> 1), _F32)
                half = d2 * 0.5
                for _ in range(3):
                    r = r * (1.5 - half * r * r)
                dist = d2 * r

                UNROLL = 8

                def kstep(i, accs):
                    k0 = i * UNROLL
                    accs = list(accs)
                    for u in range(UNROLL):
                        k = k0 + u
                        colk = jnp.full((L,), u, jnp.int32) + k0
                        va = plsc.load_gather(a_v, [rows, colk])
                        vb = plsc.load_gather(b_v, [rows, colk])
                        vc = plsc.load_gather(c_v, [rows, colk])
                        pre = va + vb + vc + dist * w1d_v[k]
                        hp = jnp.maximum(pre, 0.0)
                        accs[u % 4] = accs[u % 4] + hp * w2_v[k]
                    return tuple(accs)

                z4 = (jnp.zeros((L,), _F32),) * 4
                a0, a1, a2, a3 = lax.fori_loop(0, H // UNROLL, kstep, z4)
                w = (a0 + a1) + (a2 + a3) + b2_v[0]
                plsc.store_scatter(pay_v, [rows, c16(0)], w * rx)
                plsc.store_scatter(pay_v, [rows, c16(1)], w * ry)
                plsc.store_scatter(pay_v, [rows, c16(2)], w * rz)
                plsc.store_scatter(pay_v, [rows, c16(3)], c16(1.0, _F32))
                eaidx = rows * DE
                for j in range(DE):
                    v = plsc.load_gather(ea_v, [eaidx + j])
                    plsc.store_scatter(pay_v, [rows, c16(4 + j)], v)

            pltpu.sync_copy(pay_v, acc_sh.at[dst_i], add=True)
            return carry

        lax.fori_loop(0, NCHUNK, chunk, 0)
        plsc.subcore_barrier()

        def _drain(j, carry):
            pltpu.sync_copy(acc_sh.at[pl.ds(r0 + j * K, K)], pay_v)
            pltpu.sync_copy(pay_v, acc_o.at[cid, pl.ds(r0 + j * K, K)])
            return carry
        lax.fori_loop(0, ROWS // K, _drain, 0)

    kern = pl.kernel(
        body,
        out_type=[jax.ShapeDtypeStruct((NC, NP, H), _F32)],
        mesh=plsc.VectorSubcoreMesh(core_axis_name="c", subcore_axis_name="s"),
        scratch_types=[
            pltpu.VMEM((K,), jnp.int32),
            pltpu.VMEM((K,), jnp.int32),
            pltpu.VMEM((K, H), _F32),
            pltpu.VMEM((K, H), _F32),
            pltpu.VMEM((K, H), _F32),
            pltpu.VMEM((K, H), _F32),
            pltpu.VMEM((K * DE,), _F32),
            pltpu.VMEM((K,), _F32),
            pltpu.VMEM((K,), _F32),
            pltpu.VMEM((K,), _F32),
            pltpu.VMEM((K,), _F32),
            pltpu.VMEM((K,), _F32),
            pltpu.VMEM((K,), _F32),
            pltpu.VMEM((H,), _F32),
            pltpu.SMEM((H,), _F32),
            pltpu.SMEM((H,), _F32),
            pltpu.SMEM((1,), _F32),
            pltpu.VMEM_SHARED((NP, H), _F32),
            pltpu.SemaphoreType.DMA,
            pltpu.SemaphoreType.DMA,
            pltpu.SemaphoreType.DMA,
        ],
        compiler_params=pltpu.CompilerParams(needs_layout_passes=False),
    )
    return kern


def kernel(x, edge_index, edge_attr, pos, W1c, b1c, W2c, b2c,
           W1n, b1n, W2n, b2n):
    N, DF = x.shape
    E = edge_index.shape[1]
    DE = edge_attr.shape[1]
    H = W1c.shape[1]
    NC, NS = 2, 16
    K = 80
    assert E % (NC * NS * K) == 0 and N % NS == 0

    W1c_src = W1c[:DF]
    W1c_dst = W1c[DF:2 * DF]
    W1c_e = W1c[2 * DF:2 * DF + DE]
    w1d = W1c[2 * DF + DE]
    b1c2 = b1c.reshape(1, H)
    w2c = W2c.reshape(H)
    pos_pad = jnp.pad(pos, ((0, 0), (0, 13)))
    posx = pos[:, 0]
    posy = pos[:, 1]
    posz = pos[:, 2]
    ea_flat = edge_attr.reshape(-1)
    src = edge_index[0]
    dst = edge_index[1]

    NB = 1000
    A, B = pl.pallas_call(
        _ab_body,
        grid=(N // NB,),
        in_specs=[pl.BlockSpec((NB, DF), lambda i: (i, 0)),
                  pl.BlockSpec((DF, H), lambda i: (0, 0)),
                  pl.BlockSpec((DF, H), lambda i: (0, 0))],
        out_specs=[pl.BlockSpec((NB, H), lambda i: (i, 0)),
                   pl.BlockSpec((NB, H), lambda i: (i, 0))],
        out_shape=[jax.ShapeDtypeStruct((N, H), _F32),
                   jax.ShapeDtypeStruct((N, H), _F32)],
    )(x, W1c_src, W1c_dst)

    EB = 4000
    C = pl.pallas_call(
        _c_body,
        grid=(E // EB,),
        in_specs=[pl.BlockSpec((EB, DE), lambda i: (i, 0)),
                  pl.BlockSpec((DE, H), lambda i: (0, 0)),
                  pl.BlockSpec((1, H), lambda i: (0, 0))],
        out_specs=pl.BlockSpec((EB, H), lambda i: (i, 0)),
        out_shape=jax.ShapeDtypeStruct((E, H), _F32),
    )(edge_attr, W1c_e, b1c2)

    NP = ((N + NS * K - 1) // (NS * K)) * NS * K
    sc = _build_sc_edge(N, NP, E, DF, DE, H, K, NC, NS)
    (acc,) = sc(A, B, posx, posy, posz, C, ea_flat, src, dst, w2c, w1d, b2c)

    W1n_x = W1n[:DF]
    W1n_e = W1n[DF:]
    b1n2 = b1n.reshape(1, H)
    b2n2 = b2n.reshape(1, DF)
    new_x, new_pos_pad = pl.pallas_call(
        _node_body,
        grid=(N // NB,),
        in_specs=[pl.BlockSpec((NC, NB, H), lambda i: (0, i, 0)),
                  pl.BlockSpec((NB, DF), lambda i: (i, 0)),
                  pl.BlockSpec((NB, 16), lambda i: (i, 0)),
                  pl.BlockSpec((DF, H), lambda i: (0, 0)),
                  pl.BlockSpec((DE, H), lambda i: (0, 0)),
                  pl.BlockSpec((1, H), lambda i: (0, 0)),
                  pl.BlockSpec((H, DF), lambda i: (0, 0)),
                  pl.BlockSpec((1, DF), lambda i: (0, 0))],
        out_specs=[pl.BlockSpec((NB, DF), lambda i: (i, 0)),
                   pl.BlockSpec((NB, 16), lambda i: (i, 0))],
        out_shape=[jax.ShapeDtypeStruct((N, DF), _F32),
                   jax.ShapeDtypeStruct((N, 16), _F32)],
    )(acc, x, pos_pad, W1n_x, W1n_e, b1n2, W2n, b2n2)

    return (new_x, new_pos_pad[:, :3])


# speedup vs baseline: 1.9103x; 1.0002x over previous
import jax
import jax.numpy as jnp
from jax import lax
from jax.experimental import pallas as pl
from jax.experimental.pallas import tpu as pltpu
from jax.experimental.pallas import tpu_sc as plsc

_F32 = jnp.float32
_HIGH = jax.lax.Precision.HIGHEST


def _dot(a, b):
    return jnp.dot(a, b, preferred_element_type=_F32, precision=_HIGH)


def _ab_body(x_ref, wa_ref, wb_ref, a_ref, b_ref):
    xb = x_ref[...]
    a_ref[...] = _dot(xb, wa_ref[...])
    b_ref[...] = _dot(xb, wb_ref[...])


def _c_body(ea_ref, we_ref, b1_ref, c_ref):
    c_ref[...] = _dot(ea_ref[...], we_ref[...]) + b1_ref[...]


def _node_body(acc_ref, x_ref, pp_ref, w1x_ref, w1e_ref, b1_ref,
               w2_ref, b2_ref, nx_ref, np_ref):
    a = acc_ref[0] + acc_ref[1]
    denom = jnp.maximum(a[:, 3:4], 1.0)
    np_ref[...] = pp_ref[...] + a[:, 0:16] / denom
    agg = a[:, 4:20] / denom
    hmid = jnp.maximum(
        _dot(x_ref[...], w1x_ref[...]) + _dot(agg, w1e_ref[...]) + b1_ref[...],
        0.0)
    nx_ref[...] = _dot(hmid, w2_ref[...]) + b2_ref[...]


def _build_sc_edge(N, NP, E, DF, DE, H, K, NC, NS):
    EPW = E // (NC * NS)
    NCHUNK = EPW // K
    ROWS = NP // NS
    L = 16
    PAY = H
    assert K % L == 0 and ROWS % K == 0

    def c16(v, dtype=jnp.int32):
        return jnp.full((L,), v, dtype)

    def body(a_h, b_h, px_h, py_h, pz_h, c_h, ea_h, src_h, dst_h,
             w2_h, w1d_h, b2_h, acc_o,
             src_i, dst_i, a_v, b_v, c_v, pay_v, ea_v,
             psx, psy, psz, pdx, pdy, pdz,
             wtmp, w2_v, w1d_v, b2_v, acc_sh,
             sem_a, sem_b, sem_p):
        cid = lax.axis_index("c")
        sid = lax.axis_index("s")
        wid = cid * NS + sid

        zf = jnp.zeros((L,), _F32)

        def _zero_pay(i, carry):
            for c in range(PAY // L):
                pay_v[i, pl.ds(c * L, L)] = zf
            return carry
        lax.fori_loop(0, K, _zero_pay, 0)

        r0 = sid * ROWS

        def _zero_acc(j, carry):
            pltpu.sync_copy(pay_v, acc_sh.at[pl.ds(r0 + j * K, K)])
            return carry
        lax.fori_loop(0, ROWS // K, _zero_acc, 0)

        pltpu.sync_copy(w2_h, wtmp)
        for gg in range(H // L):
            v = wtmp[pl.ds(gg * L, L)]
            for j in range(L):
                w2_v[gg * L + j] = v[j]
        pltpu.sync_copy(w1d_h, wtmp)
        for gg in range(H // L):
            v = wtmp[pl.ds(gg * L, L)]
            for j in range(L):
                w1d_v[gg * L + j] = v[j]
        pltpu.sync_copy(b2_h, wtmp.at[pl.ds(0, 1)])
        vb2 = wtmp[pl.ds(0, L)]
        b2_v[0] = vb2[0]
        plsc.subcore_barrier()

        lanes = jnp.arange(L, dtype=jnp.int32)
        ebase = wid * EPW

        def chunk(ci, carry):
            eb = ebase + ci * K
            pltpu.sync_copy(src_h.at[pl.ds(eb, K)], src_i)
            pltpu.sync_copy(dst_h.at[pl.ds(eb, K)], dst_i)
            cp_a = pltpu.async_copy(a_h.at[src_i], a_v, sem_a)
            cp_b = pltpu.async_copy(b_h.at[dst_i], b_v, sem_b)
            cp1 = pltpu.async_copy(px_h.at[src_i], psx, sem_p)
            cp2 = pltpu.async_copy(py_h.at[src_i], psy, sem_p)
            cp3 = pltpu.async_copy(pz_h.at[src_i], psz, sem_p)
            cp4 = pltpu.async_copy(px_h.at[dst_i], pdx, sem_p)
            cp5 = pltpu.async_copy(py_h.at[dst_i], pdy, sem_p)
            cp6 = pltpu.async_copy(pz_h.at[dst_i], pdz, sem_p)
            pltpu.sync_copy(c_h.at[pl.ds(eb, K)], c_v)
            pltpu.sync_copy(ea_h.at[pl.ds(eb * DE, K * DE)], ea_v)
            cp1.wait()
            cp2.wait()
            cp3.wait()
            cp4.wait()
            cp5.wait()
            cp6.wait()
            cp_a.wait()
            cp_b.wait()

            for g in range(K // L):
                rows = lanes + (g * L)
                gl = g * L
                rx = pdx[pl.ds(gl, L)] - psx[pl.ds(gl, L)]
                ry = pdy[pl.ds(gl, L)] - psy[pl.ds(gl, L)]
                rz = pdz[pl.ds(gl, L)] - psz[pl.ds(gl, L)]
                d2 = rx * rx + ry * ry + rz * rz
                ib = plsc.bitcast(d2, jnp.int32)
                r = plsc.bitcast(c16(0x5F3759DF) - (ib >> 1), _F32)
                half = d2 * 0.5
                for _ in range(3):
                    r = r * (1.5 - half * r * r)
                dist = d2 * r

                UNROLL = 8

                def kstep(i, accs):
                    k0 = i * UNROLL
                    accs = list(accs)
                    for u in range(UNROLL):
                        k = k0 + u
                        colk = jnp.full((L,), u, jnp.int32) + k0
                        va = plsc.load_gather(a_v, [rows, colk])
                        vb = plsc.load_gather(b_v, [rows, colk])
                        vc = plsc.load_gather(c_v, [rows, colk])
                        pre = va + vb + vc + dist * w1d_v[k]
                        hp = jnp.maximum(pre, 0.0)
                        accs[u % 4] = accs[u % 4] + hp * w2_v[k]
                    return tuple(accs)

                z4 = (jnp.zeros((L,), _F32),) * 4
                a0, a1, a2, a3 = lax.fori_loop(0, H // UNROLL, kstep, z4)
                w = (a0 + a1) + (a2 + a3) + b2_v[0]
                plsc.store_scatter(pay_v, [rows, c16(0)], w * rx)
                plsc.store_scatter(pay_v, [rows, c16(1)], w * ry)
                plsc.store_scatter(pay_v, [rows, c16(2)], w * rz)
                plsc.store_scatter(pay_v, [rows, c16(3)], c16(1.0, _F32))
                eaidx = rows * DE
                for j in range(DE):
                    v = plsc.load_gather(ea_v, [eaidx + j])
                    plsc.store_scatter(pay_v, [rows, c16(4 + j)], v)

            pltpu.sync_copy(pay_v, acc_sh.at[dst_i], add=True)
            return carry

        lax.fori_loop(0, NCHUNK, chunk, 0)
        plsc.subcore_barrier()

        def _drain(j, carry):
            pltpu.sync_copy(acc_sh.at[pl.ds(r0 + j * K, K)], pay_v)
            pltpu.sync_copy(pay_v, acc_o.at[cid, pl.ds(r0 + j * K, K)])
            return carry
        lax.fori_loop(0, ROWS // K, _drain, 0)

    kern = pl.kernel(
        body,
        out_type=[jax.ShapeDtypeStruct((NC, NP, PAY), _F32)],
        mesh=plsc.VectorSubcoreMesh(core_axis_name="c", subcore_axis_name="s"),
        scratch_types=[
            pltpu.VMEM((K,), jnp.int32),
            pltpu.VMEM((K,), jnp.int32),
            pltpu.VMEM((K, H), _F32),
            pltpu.VMEM((K, H), _F32),
            pltpu.VMEM((K, H), _F32),
            pltpu.VMEM((K, PAY), _F32),
            pltpu.VMEM((K * DE,), _F32),
            pltpu.VMEM((K,), _F32),
            pltpu.VMEM((K,), _F32),
            pltpu.VMEM((K,), _F32),
            pltpu.VMEM((K,), _F32),
            pltpu.VMEM((K,), _F32),
            pltpu.VMEM((K,), _F32),
            pltpu.VMEM((H,), _F32),
            pltpu.SMEM((H,), _F32),
            pltpu.SMEM((H,), _F32),
            pltpu.SMEM((1,), _F32),
            pltpu.VMEM_SHARED((NP, PAY), _F32),
            pltpu.SemaphoreType.DMA,
            pltpu.SemaphoreType.DMA,
            pltpu.SemaphoreType.DMA,
        ],
        compiler_params=pltpu.CompilerParams(needs_layout_passes=False),
    )
    return kern


def kernel(x, edge_index, edge_attr, pos, W1c, b1c, W2c, b2c,
           W1n, b1n, W2n, b2n):
    N, DF = x.shape
    E = edge_index.shape[1]
    DE = edge_attr.shape[1]
    H = W1c.shape[1]
    NC, NS = 2, 16
    K = 80
    NP = ((N + NS * K - 1) // (NS * K)) * NS * K
    assert E % (NC * NS * K) == 0

    W1c_src = W1c[:DF]
    W1c_dst = W1c[DF:2 * DF]
    W1c_e = W1c[2 * DF:2 * DF + DE]
    w1d = W1c[2 * DF + DE]
    b1c2 = b1c.reshape(1, H)
    w2c = W2c.reshape(H)
    pos_pad = jnp.pad(pos, ((0, 0), (0, 13)))
    posx = pos[:, 0]
    posy = pos[:, 1]
    posz = pos[:, 2]
    ea_flat = edge_attr.reshape(-1)
    src = edge_index[0]
    dst = edge_index[1]

    NB = 1000
    A, B = pl.pallas_call(
        _ab_body,
        grid=(N // NB,),
        in_specs=[pl.BlockSpec((NB, DF), lambda i: (i, 0)),
                  pl.BlockSpec((DF, H), lambda i: (0, 0)),
                  pl.BlockSpec((DF, H), lambda i: (0, 0))],
        out_specs=[pl.BlockSpec((NB, H), lambda i: (i, 0)),
                   pl.BlockSpec((NB, H), lambda i: (i, 0))],
        out_shape=[jax.ShapeDtypeStruct((N, H), _F32),
                   jax.ShapeDtypeStruct((N, H), _F32)],
    )(x, W1c_src, W1c_dst)

    EB = 4000
    C = pl.pallas_call(
        _c_body,
        grid=(E // EB,),
        in_specs=[pl.BlockSpec((EB, DE), lambda i: (i, 0)),
                  pl.BlockSpec((DE, H), lambda i: (0, 0)),
                  pl.BlockSpec((1, H), lambda i: (0, 0))],
        out_specs=pl.BlockSpec((EB, H), lambda i: (i, 0)),
        out_shape=jax.ShapeDtypeStruct((E, H), _F32),
    )(edge_attr, W1c_e, b1c2)

    sc = _build_sc_edge(N, NP, E, DF, DE, H, K, NC, NS)
    (acc,) = sc(A, B, posx, posy, posz, C, ea_flat, src, dst, w2c, w1d, b2c)

    W1n_x = W1n[:DF]
    W1n_e = W1n[DF:]
    b1n2 = b1n.reshape(1, H)
    b2n2 = b2n.reshape(1, DF)
    new_x, new_pos_pad = pl.pallas_call(
        _node_body,
        grid=(N // NB,),
        in_specs=[pl.BlockSpec((NC, NB, H), lambda i: (0, i, 0)),
                  pl.BlockSpec((NB, DF), lambda i: (i, 0)),
                  pl.BlockSpec((NB, 16), lambda i: (i, 0)),
                  pl.BlockSpec((DF, H), lambda i: (0, 0)),
                  pl.BlockSpec((DE, H), lambda i: (0, 0)),
                  pl.BlockSpec((1, H), lambda i: (0, 0)),
                  pl.BlockSpec((H, DF), lambda i: (0, 0)),
                  pl.BlockSpec((1, DF), lambda i: (0, 0))],
        out_specs=[pl.BlockSpec((NB, DF), lambda i: (i, 0)),
                   pl.BlockSpec((NB, 16), lambda i: (i, 0))],
        out_shape=[jax.ShapeDtypeStruct((N, DF), _F32),
                   jax.ShapeDtypeStruct((N, 16), _F32)],
    )(acc, x, pos_pad, W1n_x, W1n_e, b1n2, W2n, b2n2)

    return (new_x, new_pos_pad[:, :3])


# speedup vs baseline: 1.9174x; 1.0037x over previous
import jax
import jax.numpy as jnp
from jax import lax
from jax.experimental import pallas as pl
from jax.experimental.pallas import tpu as pltpu
from jax.experimental.pallas import tpu_sc as plsc

_F32 = jnp.float32
_HIGH = jax.lax.Precision.HIGHEST


def _dot(a, b):
    return jnp.dot(a, b, preferred_element_type=_F32, precision=_HIGH)


def _ab_body(x_ref, wa_ref, wb_ref, a_ref, b_ref):
    xb = x_ref[...]
    a_ref[...] = _dot(xb, wa_ref[...])
    b_ref[...] = _dot(xb, wb_ref[...])


def _c_body(ea_ref, we_ref, b1_ref, c_ref):
    c_ref[...] = _dot(ea_ref[...], we_ref[...]) + b1_ref[...]


def _node_body(acc_ref, x_ref, pp_ref, w1x_ref, w1e_ref, b1_ref,
               w2_ref, b2_ref, nx_ref, np_ref):
    a = acc_ref[0] + acc_ref[1]
    denom = jnp.maximum(a[:, 3:4], 1.0)
    np_ref[...] = pp_ref[...] + a[:, 0:16] / denom
    agg = a[:, 4:20] / denom
    hmid = jnp.maximum(
        _dot(x_ref[...], w1x_ref[...]) + _dot(agg, w1e_ref[...]) + b1_ref[...],
        0.0)
    nx_ref[...] = _dot(hmid, w2_ref[...]) + b2_ref[...]


def _build_sc_edge(N, NP, E, DF, DE, H, K, NC, NS):
    EPW = E // (NC * NS)
    NCHUNK = EPW // K
    ROWS = NP // NS
    L = 16
    PAY = H
    assert K % L == 0 and ROWS % K == 0

    def c16(v, dtype=jnp.int32):
        return jnp.full((L,), v, dtype)

    def body(a_h, b_h, px_h, py_h, pz_h, c_h, ea_h, src_h, dst_h,
             w2_h, w1d_h, b2_h, acc_o,
             src_i, dst_i, a_v, b_v, c_v, pay_v, ea_v,
             psx, psy, psz, pdx, pdy, pdz,
             wtmp, w2_v, w1d_v, b2_v, acc_sh,
             sem_a, sem_b, sem_p):
        cid = lax.axis_index("c")
        sid = lax.axis_index("s")
        wid = cid * NS + sid

        zf = jnp.zeros((L,), _F32)

        def _zero_pay(i, carry):
            for c in range(PAY // L):
                pay_v[i, pl.ds(c * L, L)] = zf
            return carry
        lax.fori_loop(0, K, _zero_pay, 0)

        r0 = sid * ROWS

        def _zero_acc(j, carry):
            pltpu.sync_copy(pay_v, acc_sh.at[pl.ds(r0 + j * K, K)])
            return carry
        lax.fori_loop(0, ROWS // K, _zero_acc, 0)

        pltpu.sync_copy(w2_h, wtmp)
        for gg in range(H // L):
            v = wtmp[pl.ds(gg * L, L)]
            for j in range(L):
                w2_v[gg * L + j] = v[j]
        pltpu.sync_copy(w1d_h, wtmp)
        for gg in range(H // L):
            v = wtmp[pl.ds(gg * L, L)]
            for j in range(L):
                w1d_v[gg * L + j] = v[j]
        pltpu.sync_copy(b2_h, wtmp.at[pl.ds(0, 1)])
        vb2 = wtmp[pl.ds(0, L)]
        b2_v[0] = vb2[0]
        plsc.subcore_barrier()

        lanes = jnp.arange(L, dtype=jnp.int32)
        ebase = wid * EPW

        def chunk(ci, carry):
            eb = ebase + ci * K
            pltpu.sync_copy(src_h.at[pl.ds(eb, K)], src_i)
            pltpu.sync_copy(dst_h.at[pl.ds(eb, K)], dst_i)
            cp_a = pltpu.async_copy(a_h.at[src_i], a_v, sem_a)
            cp_b = pltpu.async_copy(b_h.at[dst_i], b_v, sem_b)
            cp1 = pltpu.async_copy(px_h.at[src_i], psx, sem_p)
            cp2 = pltpu.async_copy(py_h.at[src_i], psy, sem_p)
            cp3 = pltpu.async_copy(pz_h.at[src_i], psz, sem_p)
            cp4 = pltpu.async_copy(px_h.at[dst_i], pdx, sem_p)
            cp5 = pltpu.async_copy(py_h.at[dst_i], pdy, sem_p)
            cp6 = pltpu.async_copy(pz_h.at[dst_i], pdz, sem_p)
            pltpu.sync_copy(c_h.at[pl.ds(eb, K)], c_v)
            pltpu.sync_copy(ea_h.at[pl.ds(eb * DE, K * DE)], ea_v)
            cp1.wait()
            cp2.wait()
            cp3.wait()
            cp4.wait()
            cp5.wait()
            cp6.wait()
            cp_a.wait()
            cp_b.wait()

            for g in range(K // L):
                rows = lanes + (g * L)
                gl = g * L
                rx = pdx[pl.ds(gl, L)] - psx[pl.ds(gl, L)]
                ry = pdy[pl.ds(gl, L)] - psy[pl.ds(gl, L)]
                rz = pdz[pl.ds(gl, L)] - psz[pl.ds(gl, L)]
                d2 = rx * rx + ry * ry + rz * rz
                ib = plsc.bitcast(d2, jnp.int32)
                r = plsc.bitcast(c16(0x5F3759DF) - (ib >> 1), _F32)
                half = d2 * 0.5
                for _ in range(3):
                    r = r * (1.5 - half * r * r)
                dist = d2 * r

                UNROLL = 16

                def kstep(i, accs):
                    k0 = i * UNROLL
                    accs = list(accs)
                    for u in range(UNROLL):
                        k = k0 + u
                        colk = jnp.full((L,), u, jnp.int32) + k0
                        va = plsc.load_gather(a_v, [rows, colk])
                        vb = plsc.load_gather(b_v, [rows, colk])
                        vc = plsc.load_gather(c_v, [rows, colk])
                        pre = va + vb + vc + dist * w1d_v[k]
                        hp = jnp.maximum(pre, 0.0)
                        accs[u % 4] = accs[u % 4] + hp * w2_v[k]
                    return tuple(accs)

                z4 = (jnp.zeros((L,), _F32),) * 4
                a0, a1, a2, a3 = lax.fori_loop(0, H // UNROLL, kstep, z4)
                w = (a0 + a1) + (a2 + a3) + b2_v[0]
                plsc.store_scatter(pay_v, [rows, c16(0)], w * rx)
                plsc.store_scatter(pay_v, [rows, c16(1)], w * ry)
                plsc.store_scatter(pay_v, [rows, c16(2)], w * rz)
                plsc.store_scatter(pay_v, [rows, c16(3)], c16(1.0, _F32))
                eaidx = rows * DE
                for j in range(DE):
                    v = plsc.load_gather(ea_v, [eaidx + j])
                    plsc.store_scatter(pay_v, [rows, c16(4 + j)], v)

            pltpu.sync_copy(pay_v, acc_sh.at[dst_i], add=True)
            return carry

        lax.fori_loop(0, NCHUNK, chunk, 0)
        plsc.subcore_barrier()

        def _drain(j, carry):
            pltpu.sync_copy(acc_sh.at[pl.ds(r0 + j * K, K)], pay_v)
            pltpu.sync_copy(pay_v, acc_o.at[cid, pl.ds(r0 + j * K, K)])
            return carry
        lax.fori_loop(0, ROWS // K, _drain, 0)

    kern = pl.kernel(
        body,
        out_type=[jax.ShapeDtypeStruct((NC, NP, PAY), _F32)],
        mesh=plsc.VectorSubcoreMesh(core_axis_name="c", subcore_axis_name="s"),
        scratch_types=[
            pltpu.VMEM((K,), jnp.int32),
            pltpu.VMEM((K,), jnp.int32),
            pltpu.VMEM((K, H), _F32),
            pltpu.VMEM((K, H), _F32),
            pltpu.VMEM((K, H), _F32),
            pltpu.VMEM((K, PAY), _F32),
            pltpu.VMEM((K * DE,), _F32),
            pltpu.VMEM((K,), _F32),
            pltpu.VMEM((K,), _F32),
            pltpu.VMEM((K,), _F32),
            pltpu.VMEM((K,), _F32),
            pltpu.VMEM((K,), _F32),
            pltpu.VMEM((K,), _F32),
            pltpu.VMEM((H,), _F32),
            pltpu.SMEM((H,), _F32),
            pltpu.SMEM((H,), _F32),
            pltpu.SMEM((1,), _F32),
            pltpu.VMEM_SHARED((NP, PAY), _F32),
            pltpu.SemaphoreType.DMA,
            pltpu.SemaphoreType.DMA,
            pltpu.SemaphoreType.DMA,
        ],
        compiler_params=pltpu.CompilerParams(needs_layout_passes=False),
    )
    return kern


def kernel(x, edge_index, edge_attr, pos, W1c, b1c, W2c, b2c,
           W1n, b1n, W2n, b2n):
    N, DF = x.shape
    E = edge_index.shape[1]
    DE = edge_attr.shape[1]
    H = W1c.shape[1]
    NC, NS = 2, 16
    K = 80
    NP = ((N + NS * K - 1) // (NS * K)) * NS * K
    assert E % (NC * NS * K) == 0

    W1c_src = W1c[:DF]
    W1c_dst = W1c[DF:2 * DF]
    W1c_e = W1c[2 * DF:2 * DF + DE]
    w1d = W1c[2 * DF + DE]
    b1c2 = b1c.reshape(1, H)
    w2c = W2c.reshape(H)
    pos_pad = jnp.pad(pos, ((0, 0), (0, 13)))
    posx = pos[:, 0]
    posy = pos[:, 1]
    posz = pos[:, 2]
    ea_flat = edge_attr.reshape(-1)
    src = edge_index[0]
    dst = edge_index[1]

    NB = 1000
    A, B = pl.pallas_call(
        _ab_body,
        grid=(N // NB,),
        in_specs=[pl.BlockSpec((NB, DF), lambda i: (i, 0)),
                  pl.BlockSpec((DF, H), lambda i: (0, 0)),
                  pl.BlockSpec((DF, H), lambda i: (0, 0))],
        out_specs=[pl.BlockSpec((NB, H), lambda i: (i, 0)),
                   pl.BlockSpec((NB, H), lambda i: (i, 0))],
        out_shape=[jax.ShapeDtypeStruct((N, H), _F32),
                   jax.ShapeDtypeStruct((N, H), _F32)],
    )(x, W1c_src, W1c_dst)

    EB = 4000
    C = pl.pallas_call(
        _c_body,
        grid=(E // EB,),
        in_specs=[pl.BlockSpec((EB, DE), lambda i: (i, 0)),
                  pl.BlockSpec((DE, H), lambda i: (0, 0)),
                  pl.BlockSpec((1, H), lambda i: (0, 0))],
        out_specs=pl.BlockSpec((EB, H), lambda i: (i, 0)),
        out_shape=jax.ShapeDtypeStruct((E, H), _F32),
    )(edge_attr, W1c_e, b1c2)

    sc = _build_sc_edge(N, NP, E, DF, DE, H, K, NC, NS)
    (acc,) = sc(A, B, posx, posy, posz, C, ea_flat, src, dst, w2c, w1d, b2c)

    W1n_x = W1n[:DF]
    W1n_e = W1n[DF:]
    b1n2 = b1n.reshape(1, H)
    b2n2 = b2n.reshape(1, DF)
    new_x, new_pos_pad = pl.pallas_call(
        _node_body,
        grid=(N // NB,),
        in_specs=[pl.BlockSpec((NC, NB, H), lambda i: (0, i, 0)),
                  pl.BlockSpec((NB, DF), lambda i: (i, 0)),
                  pl.BlockSpec((NB, 16), lambda i: (i, 0)),
                  pl.BlockSpec((DF, H), lambda i: (0, 0)),
                  pl.BlockSpec((DE, H), lambda i: (0, 0)),
                  pl.BlockSpec((1, H), lambda i: (0, 0)),
                  pl.BlockSpec((H, DF), lambda i: (0, 0)),
                  pl.BlockSpec((1, DF), lambda i: (0, 0))],
        out_specs=[pl.BlockSpec((NB, DF), lambda i: (i, 0)),
                   pl.BlockSpec((NB, 16), lambda i: (i, 0))],
        out_shape=[jax.ShapeDtypeStruct((N, DF), _F32),
                   jax.ShapeDtypeStruct((N, 16), _F32)],
    )(acc, x, pos_pad, W1n_x, W1n_e, b1n2, W2n, b2n2)

    return (new_x, new_pos_pad[:, :3])
